# baseline (speedup 1.0000x reference)
import jax
import jax.numpy as jnp
from jax.experimental import pallas as pl
from jax.experimental.pallas import tpu as pltpu

_CHUNK = 128
_SLOTS = 4
_AHEAD = 3
_RPB = 2


def _gather_t_kernel(ids_ref, w_hbm, o_ref, scratch, sems):
    b = pl.program_id(0)
    L = o_ref.shape[2]
    cpb = o_ref.shape[0] * L // _CHUNK
    total_chunks = ids_ref.shape[1] // _CHUNK

    def issue_slice(tbase, slot_off, n):
        for j in range(n):
            row = ids_ref[0, tbase + j]
            pltpu.make_async_copy(
                w_hbm.at[pl.ds(row, 1), :],
                scratch.at[pl.ds(slot_off + j, 1), :],
                sems.at[slot_off // _CHUNK],
            ).start()

    @pl.when(b == 0)
    def _():
        for c in range(_AHEAD):
            issue_slice(c * _CHUNK, c * _CHUNK, _CHUNK)

    for k in range(cpb):
        slot = k % _SLOTS
        tgt = (k + _AHEAD) % _SLOTS
        c_fut = cpb * b + k + _AHEAD
        t_fut = c_fut * _CHUNK

        @pl.when(c_fut < total_chunks)
        def _():
            issue_slice(t_fut, tgt * _CHUNK, _CHUNK)

        pltpu.make_async_copy(
            w_hbm.at[pl.ds(0, _CHUNK), :],
            scratch.at[pl.ds(slot * _CHUNK, _CHUNK), :],
            sems.at[slot],
        ).wait()
        blk = scratch[pl.ds(slot * _CHUNK, _CHUNK), :]
        lpc = L // _CHUNK
        o_ref[k // lpc, :, pl.ds((k % lpc) * _CHUNK, _CHUNK)] = jnp.transpose(blk)


def kernel(x, weight):
    B, L = x.shape
    V, D = weight.shape
    ids = x.reshape(1, B * L).astype(jnp.int32)
    return pl.pallas_call(
        _gather_t_kernel,
        out_shape=jax.ShapeDtypeStruct((B, D, L), weight.dtype),
        grid_spec=pltpu.PrefetchScalarGridSpec(
            num_scalar_prefetch=1,
            grid=(B // _RPB,),
            in_specs=[pl.BlockSpec(memory_space=pl.ANY)],
            out_specs=pl.BlockSpec((_RPB, D, L), lambda b, ids: (b, 0, 0)),
            scratch_shapes=[
                pltpu.VMEM((_SLOTS * _CHUNK, D), weight.dtype),
                pltpu.SemaphoreType.DMA((_SLOTS,)),
            ],
        ),
        compiler_params=pltpu.CompilerParams(
            dimension_semantics=("arbitrary",),
            disable_bounds_checks=True,
        ),
    )(ids, weight)


# speedup vs baseline: 1.0059x; 1.0059x over previous
import jax
import jax.numpy as jnp
from jax.experimental import pallas as pl
from jax.experimental.pallas import tpu as pltpu

_CHUNK = 128
_SLOTS = 4
_AHEAD = 3


def _gather_t_kernel(ids_ref, w_hbm, o_ref, scratch, sems):
    b = pl.program_id(0)
    L = o_ref.shape[1]
    cpb = L // _CHUNK
    total_chunks = ids_ref.shape[1] // _CHUNK

    def issue_slice(tbase, slot_off, n):
        for j in range(n):
            row = ids_ref[0, tbase + j]
            pltpu.make_async_copy(
                w_hbm.at[pl.ds(row, 1), :],
                scratch.at[pl.ds(slot_off + j, 1), :],
                sems.at[slot_off // _CHUNK],
            ).start()

    @pl.when(b == 0)
    def _():
        for c in range(_AHEAD):
            issue_slice(c * _CHUNK, c * _CHUNK, _CHUNK)

    for k in range(cpb):
        slot = k
        tgt = (k + _AHEAD) % _SLOTS
        c_fut = cpb * b + k + _AHEAD
        t_fut = c_fut * _CHUNK

        @pl.when(c_fut < total_chunks)
        def _():
            issue_slice(t_fut, tgt * _CHUNK, _CHUNK)

        pltpu.make_async_copy(
            w_hbm.at[pl.ds(0, _CHUNK), :],
            scratch.at[pl.ds(slot * _CHUNK, _CHUNK), :],
            sems.at[slot],
        ).wait()
        blk = scratch[pl.ds(slot * _CHUNK, _CHUNK), :]
        o_ref[:, pl.ds(k * _CHUNK, _CHUNK)] = jnp.transpose(blk)


def kernel(x, weight):
    B, L = x.shape
    V, D = weight.shape
    ids = x.reshape(1, B * L).astype(jnp.int32)
    return pl.pallas_call(
        _gather_t_kernel,
        out_shape=jax.ShapeDtypeStruct((B, D, L), weight.dtype),
        grid_spec=pltpu.PrefetchScalarGridSpec(
            num_scalar_prefetch=1,
            grid=(B,),
            in_specs=[pl.BlockSpec(memory_space=pl.ANY)],
            out_specs=pl.BlockSpec((None, D, L), lambda b, ids: (b, 0, 0)),
            scratch_shapes=[
                pltpu.VMEM((_SLOTS * _CHUNK, D), weight.dtype),
                pltpu.SemaphoreType.DMA((_SLOTS,)),
            ],
        ),
        compiler_params=pltpu.CompilerParams(
            dimension_semantics=("arbitrary",),
            disable_bounds_checks=True,
        ),
    )(ids, weight)


# speedup vs baseline: 1.0067x; 1.0008x over previous
import jax
import jax.numpy as jnp
from jax.experimental import pallas as pl
from jax.experimental.pallas import tpu as pltpu

_CHUNK = 128
_SLOTS = 4
_AHEAD = 3


def _gather_t_kernel(ids_ref, w_hbm, o_ref, scratch, sems):
    b = pl.program_id(0)
    L = o_ref.shape[1]
    cpb = L // _CHUNK
    total_chunks = ids_ref.shape[0] // _CHUNK

    def issue_slice(tbase, slot_off, n):
        for j in range(n):
            row = ids_ref[tbase + j]
            pltpu.make_async_copy(
                w_hbm.at[pl.ds(row, 1), :],
                scratch.at[pl.ds(slot_off + j, 1), :],
                sems.at[slot_off // _CHUNK],
            ).start()

    @pl.when(b == 0)
    def _():
        for c in range(_AHEAD):
            issue_slice(c * _CHUNK, c * _CHUNK, _CHUNK)

    for k in range(cpb):
        slot = k
        tgt = (k + _AHEAD) % _SLOTS
        c_fut = cpb * b + k + _AHEAD
        t_fut = c_fut * _CHUNK

        @pl.when(c_fut < total_chunks)
        def _():
            issue_slice(t_fut, tgt * _CHUNK, _CHUNK)

        pltpu.make_async_copy(
            w_hbm.at[pl.ds(0, _CHUNK), :],
            scratch.at[pl.ds(slot * _CHUNK, _CHUNK), :],
            sems.at[slot],
        ).wait()
        blk = scratch[pl.ds(slot * _CHUNK, _CHUNK), :]
        o_ref[:, pl.ds(k * _CHUNK, _CHUNK)] = jnp.transpose(blk)


def kernel(x, weight):
    B, L = x.shape
    V, D = weight.shape
    ids = x.reshape(B * L).astype(jnp.int32)
    return pl.pallas_call(
        _gather_t_kernel,
        out_shape=jax.ShapeDtypeStruct((B, D, L), weight.dtype),
        grid_spec=pltpu.PrefetchScalarGridSpec(
            num_scalar_prefetch=1,
            grid=(B,),
            in_specs=[pl.BlockSpec(memory_space=pl.ANY)],
            out_specs=pl.BlockSpec((None, D, L), lambda b, ids: (b, 0, 0)),
            scratch_shapes=[
                pltpu.VMEM((_SLOTS * _CHUNK, D), weight.dtype),
                pltpu.SemaphoreType.DMA((_SLOTS,)),
            ],
        ),
        compiler_params=pltpu.CompilerParams(
            dimension_semantics=("arbitrary",),
            disable_bounds_checks=True,
        ),
    )(ids, weight)
